# baseline (speedup 1.0000x reference)
import functools

import jax
import jax.numpy as jnp
from jax import lax
from jax.experimental import pallas as pl
from jax.experimental.pallas import tpu as pltpu
from jax.experimental.pallas import tpu_sc as plsc

_VOCAB = 1000000
_E = 64
_B = 4096
_L = 200
_LANES = 16
_NC = 2
_NS = 16
_NW = _NC * _NS
_BPW = _B // _NW
_NBUF = 4
_LP = 208
_CHUNKS = ((0, 128), (128, 72))


def _bow_body(data_hbm, len_hbm, table_hbm, out_hbm,
              idx_v, len_v, pairs_v, rows_v, outb_v, sem_g, sem_o):
  wid = lax.axis_index("s") * _NC + lax.axis_index("c")
  base = wid * _BPW

  pltpu.sync_copy(data_hbm.at[pl.ds(base * _L, _BPW * _L)],
                  idx_v.at[pl.ds(0, _BPW * _L)])
  pltpu.sync_copy(len_hbm.at[pl.ds(base, _BPW)], len_v.at[pl.ds(0, _BPW)])

  def prep_pairs(b, buf):
    for k in range(13):
      v = idx_v[pl.ds(b * _L + k * _LANES, _LANES)]
      pairs_v[buf, pl.ds(k * _LANES, _LANES)] = lax.shift_right_logical(v, 1)

  def start_gathers(b, buf):
    prep_pairs(b, buf)
    for off, n in _CHUNKS:
      pltpu.async_copy(
          table_hbm.at[pairs_v.at[buf, pl.ds(off, n)]],
          rows_v.at[buf, pl.ds(off, n)],
          sem_g[buf])

  def wait_gathers(buf):
    pltpu.make_async_copy(
        table_hbm.at[pl.ds(0, _L)], rows_v.at[buf, pl.ds(0, _L)],
        sem_g[buf]).wait()

  def compute(b, buf):
    def fold(i, accs):
      l = 2 * i
      out = []
      for c in range(4):
        s = pl.ds(c * _LANES, _LANES)
        out.append(accs[c] + rows_v[buf, l, s])
        out.append(accs[c + 4] + rows_v[buf, l + 1, s])
      return (out[0], out[2], out[4], out[6], out[1], out[3], out[5], out[7])

    zero = jnp.zeros((_LANES,), jnp.float32)
    accs = lax.fori_loop(0, _L // 2, fold, (zero,) * 8, unroll=5)

    lenf = len_v[pl.ds(b, _LANES)][0].astype(jnp.float32)
    recip = jnp.full((_LANES,), 1.0, jnp.float32) / lenf
    for c in range(4):
      outb_v[buf, 0, pl.ds(c * _LANES, _LANES)] = (accs[c] + accs[c + 4]) * recip

  for j in range(_NBUF - 1):
    start_gathers(j, j)

  def step(b, buf):
    wait_gathers(buf)

    @pl.when(b >= _NBUF)
    def _():
      pltpu.make_async_copy(
          outb_v.at[buf], out_hbm.at[pl.ds(0, 1)], sem_o[buf]).wait()

    compute(b, buf)
    pltpu.async_copy(
        outb_v.at[buf], out_hbm.at[pl.ds(base + b, 1)], sem_o[buf])

    @pl.when(b + _NBUF - 1 < _BPW)
    def _():
      start_gathers(b + _NBUF - 1, (buf + _NBUF - 1) % _NBUF)

  def outer(i, carry):
    for j in range(_NBUF):
      step(_NBUF * i + j, j)
    return carry

  lax.fori_loop(0, _BPW // _NBUF, outer, 0)

  for j in range(_NBUF):
    pltpu.make_async_copy(
        outb_v.at[j], out_hbm.at[pl.ds(0, 1)], sem_o[j]).wait()


_bow = functools.partial(
    pl.kernel,
    mesh=plsc.VectorSubcoreMesh(core_axis_name="c", subcore_axis_name="s"),
    out_type=jax.ShapeDtypeStruct((_B, _E), jnp.float32),
    scratch_types=[
        pltpu.VMEM((_BPW * _L + _LANES,), jnp.int32),
        pltpu.VMEM((_BPW + _LANES,), jnp.int32),
        pltpu.VMEM((_NBUF, _LP), jnp.int32),
        pltpu.VMEM((_NBUF, _L, 2 * _E), jnp.float32),
        pltpu.VMEM((_NBUF, 1, _E), jnp.float32),
        [pltpu.SemaphoreType.DMA] * _NBUF,
        [pltpu.SemaphoreType.DMA] * _NBUF,
    ],
    compiler_params=pltpu.CompilerParams(use_tc_tiling_on_sc=False),
)(_bow_body)


@jax.jit
def kernel(data_bl, length_b, table):
  data_flat = data_bl.reshape(_B * _L)
  len_flat = length_b.reshape(_B)
  table_pairs = table.reshape(_VOCAB // 2, 2 * _E)
  return _bow(data_flat, len_flat, table_pairs)


# speedup vs baseline: 1.0816x; 1.0816x over previous
import functools

import jax
import jax.numpy as jnp
from jax import lax
from jax.experimental import pallas as pl
from jax.experimental.pallas import tpu as pltpu
from jax.experimental.pallas import tpu_sc as plsc

_VOCAB = 1000000
_E = 64
_B = 4096
_L = 200
_LANES = 16
_NC = 2
_NS = 16
_NW = _NC * _NS
_BPW = _B // _NW
_NBUF = 4
_LP = 208
_CHUNKS = ((0, 128), (128, 72))


def _bow_body(data_hbm, len_hbm, table_hbm, out_hbm,
              idx_v, len_v, rows_v, outb_v, sem_g, sem_o):
  wid = lax.axis_index("s") * _NC + lax.axis_index("c")
  base = wid * _BPW

  pltpu.sync_copy(data_hbm.at[pl.ds(base * _L, _BPW * _L)],
                  idx_v.at[pl.ds(0, _BPW * _L)])
  pltpu.sync_copy(len_hbm.at[pl.ds(base, _BPW)], len_v.at[pl.ds(0, _BPW)])

  def start_gathers(b, buf):
    for off, n in _CHUNKS:
      pltpu.async_copy(
          table_hbm.at[idx_v.at[pl.ds(b * _L + off, n)]],
          rows_v.at[buf, pl.ds(off, n)],
          sem_g[buf])

  def wait_gathers(buf):
    pltpu.make_async_copy(
        table_hbm.at[pl.ds(0, _L)], rows_v.at[buf, pl.ds(0, _L)],
        sem_g[buf]).wait()

  def compute(b, buf):
    def fold(i, accs):
      l = 2 * i
      out = []
      for c in range(4):
        s = pl.ds(c * _LANES, _LANES)
        out.append(accs[c] + rows_v[buf, l, s])
        out.append(accs[c + 4] + rows_v[buf, l + 1, s])
      return (out[0], out[2], out[4], out[6], out[1], out[3], out[5], out[7])

    zero = jnp.zeros((_LANES,), jnp.float32)
    accs = lax.fori_loop(0, _L // 2, fold, (zero,) * 8, unroll=5)

    lenf = len_v[pl.ds(b, _LANES)][0].astype(jnp.float32)
    recip = jnp.full((_LANES,), 1.0, jnp.float32) / lenf
    for c in range(4):
      outb_v[buf, 0, pl.ds(c * _LANES, _LANES)] = (accs[c] + accs[c + 4]) * recip

  for j in range(_NBUF - 1):
    start_gathers(j, j)

  def step(b, buf):
    wait_gathers(buf)

    @pl.when(b >= _NBUF)
    def _():
      pltpu.make_async_copy(
          outb_v.at[buf], out_hbm.at[pl.ds(0, 1)], sem_o[buf]).wait()

    compute(b, buf)
    pltpu.async_copy(
        outb_v.at[buf], out_hbm.at[pl.ds(base + b, 1)], sem_o[buf])

    @pl.when(b + _NBUF - 1 < _BPW)
    def _():
      start_gathers(b + _NBUF - 1, (buf + _NBUF - 1) % _NBUF)

  def outer(i, carry):
    for j in range(_NBUF):
      step(_NBUF * i + j, j)
    return carry

  lax.fori_loop(0, _BPW // _NBUF, outer, 0)

  for j in range(_NBUF):
    pltpu.make_async_copy(
        outb_v.at[j], out_hbm.at[pl.ds(0, 1)], sem_o[j]).wait()


_bow = functools.partial(
    pl.kernel,
    mesh=plsc.VectorSubcoreMesh(core_axis_name="c", subcore_axis_name="s"),
    out_type=jax.ShapeDtypeStruct((_B, _E), jnp.float32),
    scratch_types=[
        pltpu.VMEM((_BPW * _L + _LANES,), jnp.int32),
        pltpu.VMEM((_BPW + _LANES,), jnp.int32),
        pltpu.VMEM((_NBUF, _L, 2 * _E), jnp.float32),
        pltpu.VMEM((_NBUF, 1, _E), jnp.float32),
        [pltpu.SemaphoreType.DMA] * _NBUF,
        [pltpu.SemaphoreType.DMA] * _NBUF,
    ],
    compiler_params=pltpu.CompilerParams(use_tc_tiling_on_sc=False),
)(_bow_body)


@jax.jit
def kernel(data_bl, length_b, table):
  data_flat = data_bl.reshape(_B * _L)
  len_flat = length_b.reshape(_B)
  table_pad = jnp.pad(table, ((0, 0), (0, _E)))
  return _bow(data_flat, len_flat, table_pad)
